# Initial kernel scaffold; baseline (speedup 1.0000x reference)
import jax
import jax.numpy as jnp
from jax.experimental import pallas as pl


def kernel(feat, edge_index, eps):
    raise NotImplementedError("write your pallas kernel here")


# speedup vs baseline: 5.3503x; 5.3503x over previous
import functools

import jax
import jax.numpy as jnp
from jax import lax
from jax.experimental import pallas as pl
from jax.experimental.pallas import tpu as pltpu
from jax.experimental.pallas import tpu_sc as plsc

N_NODES = 10000
N_EDGES = 320000
D_FEAT = 128

NC = 2
NS = 16
NW = NC * NS

E_PER_W = N_EDGES // NW
CHUNK = 80
CHUNKS_PER_W = E_PER_W // CHUNK
N_CHUNKS = N_EDGES // CHUNK
SLAB = 640
LAST_SLAB = N_NODES - SLAB * (NS - 1)
ZROWS = 80


def _sc_partials(feat, src3, dst3):
    mesh = plsc.VectorSubcoreMesh(
        core_axis_name="c", subcore_axis_name="s", num_cores=NC, num_subcores=NS
    )

    @functools.partial(
        pl.kernel,
        out_type=jax.ShapeDtypeStruct((NC, N_NODES, D_FEAT), jnp.float32),
        mesh=mesh,
        scratch_types=[
            pltpu.VMEM((1, CHUNK), jnp.int32),
            pltpu.VMEM((1, CHUNK), jnp.int32),
            pltpu.VMEM((CHUNK, D_FEAT), jnp.float32),
            pltpu.VMEM((ZROWS, D_FEAT), jnp.float32),
            pltpu.VMEM_SHARED((N_NODES, D_FEAT), jnp.float32),
            pltpu.SemaphoreType.DMA,
        ],
    )
    def k(feat_hbm, src_hbm, dst_hbm, part_hbm, src_v, dst_v, rows_v, zbuf,
          acc, sem):
        c = lax.axis_index("c")
        s = lax.axis_index("s")
        w = s * NC + c

        zvec = jnp.zeros((16,), jnp.float32)

        def zrow(r, carry):
            for k16 in range(D_FEAT // 16):
                zbuf[r, pl.ds(k16 * 16, 16)] = zvec
            return carry

        lax.fori_loop(0, ZROWS, zrow, 0)

        @pl.when(s < NS - 1)
        def _():
            for i in range(SLAB // ZROWS):
                pltpu.sync_copy(zbuf, acc.at[pl.ds(s * SLAB + i * ZROWS, ZROWS)])

        @pl.when(s == NS - 1)
        def _():
            for i in range(LAST_SLAB // ZROWS):
                pltpu.sync_copy(
                    zbuf, acc.at[pl.ds((NS - 1) * SLAB + i * ZROWS, ZROWS)]
                )

        plsc.subcore_barrier()

        def body(j, carry):
            jj = w * CHUNKS_PER_W + j
            pltpu.sync_copy(src_hbm.at[jj], src_v)
            pltpu.sync_copy(dst_hbm.at[jj], dst_v)
            pltpu.async_copy(feat_hbm.at[src_v.at[0]], rows_v, sem).wait()
            pltpu.sync_copy(rows_v, acc.at[dst_v.at[0]], add=True)
            return carry

        lax.fori_loop(0, CHUNKS_PER_W, body, 0)
        plsc.subcore_barrier()

        @pl.when(s < NS - 1)
        def _():
            pltpu.sync_copy(
                acc.at[pl.ds(s * SLAB, SLAB)],
                part_hbm.at[c, pl.ds(s * SLAB, SLAB)],
            )

        @pl.when(s == NS - 1)
        def _():
            pltpu.sync_copy(
                acc.at[pl.ds((NS - 1) * SLAB, LAST_SLAB)],
                part_hbm.at[c, pl.ds((NS - 1) * SLAB, LAST_SLAB)],
            )

    return k(feat, src3, dst3)


def _tc_combine_body(eps_ref, feat_ref, part_ref, o_ref):
    o_ref[...] = (1.0 + eps_ref[0]) * feat_ref[...] + part_ref[0] + part_ref[1]


def _tc_combine(eps, feat, part):
    rows = 1000
    grid = N_NODES // rows
    return pl.pallas_call(
        _tc_combine_body,
        grid=(grid,),
        in_specs=[
            pl.BlockSpec(memory_space=pltpu.SMEM),
            pl.BlockSpec((rows, D_FEAT), lambda i: (i, 0)),
            pl.BlockSpec((NC, rows, D_FEAT), lambda i: (0, i, 0)),
        ],
        out_specs=pl.BlockSpec((rows, D_FEAT), lambda i: (i, 0)),
        out_shape=jax.ShapeDtypeStruct((N_NODES, D_FEAT), jnp.float32),
    )(eps, feat, part)


@jax.jit
def kernel(feat, edge_index, eps):
    src3 = edge_index[0].astype(jnp.int32).reshape(N_CHUNKS, 1, CHUNK)
    dst3 = edge_index[1].astype(jnp.int32).reshape(N_CHUNKS, 1, CHUNK)
    part = _sc_partials(feat, src3, dst3)
    return _tc_combine(eps, feat, part)


# speedup vs baseline: 10.7671x; 2.0124x over previous
import functools

import jax
import jax.numpy as jnp
from jax import lax
from jax.experimental import pallas as pl
from jax.experimental.pallas import tpu as pltpu
from jax.experimental.pallas import tpu_sc as plsc

N_NODES = 10000
N_EDGES = 320000
D_FEAT = 128

NC = 2
NS = 16
NW = NC * NS

E_PER_W = N_EDGES // NW
CHUNK = 100
CHUNKS_PER_W = E_PER_W // CHUNK
N_CHUNKS = N_EDGES // CHUNK
SLAB = 640
LAST_SLAB = N_NODES - SLAB * (NS - 1)
ZROWS = 80


def _sc_partials(feat, src3, dst3):
    mesh = plsc.VectorSubcoreMesh(
        core_axis_name="c", subcore_axis_name="s", num_cores=NC, num_subcores=NS
    )

    @functools.partial(
        pl.kernel,
        out_type=jax.ShapeDtypeStruct((NC, N_NODES, D_FEAT), jnp.float32),
        mesh=mesh,
        scratch_types=[
            pltpu.VMEM((1, CHUNK), jnp.int32),
            pltpu.VMEM((1, CHUNK), jnp.int32),
            pltpu.VMEM((1, CHUNK), jnp.int32),
            pltpu.VMEM((1, CHUNK), jnp.int32),
            pltpu.VMEM((CHUNK, D_FEAT), jnp.float32),
            pltpu.VMEM((CHUNK, D_FEAT), jnp.float32),
            pltpu.SemaphoreType.DMA,
            pltpu.SemaphoreType.DMA,
            pltpu.SemaphoreType.DMA,
            pltpu.SemaphoreType.DMA,
            pltpu.VMEM_SHARED((N_NODES, D_FEAT), jnp.float32),
        ],
    )
    def k(feat_hbm, src_hbm, dst_hbm, part_hbm, srcb0, dstb0, srcb1, dstb1,
          rows0, rows1, sg0, sg1, si0, si1, acc):
        c = lax.axis_index("c")
        s = lax.axis_index("s")
        w = s * NC + c
        base = w * CHUNKS_PER_W

        zvec = jnp.zeros((16,), jnp.float32)

        def zrow(r, carry):
            for k16 in range(D_FEAT // 16):
                rows0[r, pl.ds(k16 * 16, 16)] = zvec
            return carry

        lax.fori_loop(0, ZROWS, zrow, 0)

        @pl.when(s < NS - 1)
        def _():
            for i in range(SLAB // ZROWS):
                pltpu.sync_copy(
                    rows0.at[pl.ds(0, ZROWS)],
                    acc.at[pl.ds(s * SLAB + i * ZROWS, ZROWS)],
                )

        @pl.when(s == NS - 1)
        def _():
            for i in range(LAST_SLAB // ZROWS):
                pltpu.sync_copy(
                    rows0.at[pl.ds(0, ZROWS)],
                    acc.at[pl.ds((NS - 1) * SLAB + i * ZROWS, ZROWS)],
                )

        pltpu.sync_copy(src_hbm.at[base], srcb0)
        pltpu.sync_copy(dst_hbm.at[base], dstb0)
        pltpu.async_copy(feat_hbm.at[srcb0.at[0]], rows0, sg0)
        pltpu.async_copy(src_hbm.at[base + 1], srcb1, si1)
        pltpu.async_copy(dst_hbm.at[base + 1], dstb1, si1)

        plsc.subcore_barrier()

        def wait_idx(bsrc, bdst, sem):
            pltpu.make_async_copy(src_hbm.at[0], bsrc, sem).wait()
            pltpu.make_async_copy(dst_hbm.at[0], bdst, sem).wait()

        def body(t, carry):
            j0 = 2 * t
            wait_idx(srcb1, dstb1, si1)
            pltpu.make_async_copy(feat_hbm.at[srcb0.at[0]], rows0, sg0).wait()
            pltpu.async_copy(feat_hbm.at[srcb1.at[0]], rows1, sg1)
            pltpu.sync_copy(rows0, acc.at[dstb0.at[0]], add=True)

            @pl.when(t < CHUNKS_PER_W // 2 - 1)
            def _():
                pltpu.async_copy(src_hbm.at[base + j0 + 2], srcb0, si0)
                pltpu.async_copy(dst_hbm.at[base + j0 + 2], dstb0, si0)
                wait_idx(srcb0, dstb0, si0)
                pltpu.async_copy(feat_hbm.at[srcb0.at[0]], rows0, sg0)

            pltpu.make_async_copy(feat_hbm.at[srcb1.at[0]], rows1, sg1).wait()
            pltpu.sync_copy(rows1, acc.at[dstb1.at[0]], add=True)

            @pl.when(t < CHUNKS_PER_W // 2 - 1)
            def _():
                pltpu.async_copy(src_hbm.at[base + j0 + 3], srcb1, si1)
                pltpu.async_copy(dst_hbm.at[base + j0 + 3], dstb1, si1)

            return carry

        lax.fori_loop(0, CHUNKS_PER_W // 2, body, 0)
        plsc.subcore_barrier()

        @pl.when(s < NS - 1)
        def _():
            pltpu.sync_copy(
                acc.at[pl.ds(s * SLAB, SLAB)],
                part_hbm.at[c, pl.ds(s * SLAB, SLAB)],
            )

        @pl.when(s == NS - 1)
        def _():
            pltpu.sync_copy(
                acc.at[pl.ds((NS - 1) * SLAB, LAST_SLAB)],
                part_hbm.at[c, pl.ds((NS - 1) * SLAB, LAST_SLAB)],
            )

    return k(feat, src3, dst3)


def _tc_combine_body(eps_ref, feat_ref, part_ref, o_ref):
    o_ref[...] = (1.0 + eps_ref[0]) * feat_ref[...] + part_ref[0] + part_ref[1]


def _tc_combine(eps, feat, part):
    rows = 1000
    grid = N_NODES // rows
    return pl.pallas_call(
        _tc_combine_body,
        grid=(grid,),
        in_specs=[
            pl.BlockSpec(memory_space=pltpu.SMEM),
            pl.BlockSpec((rows, D_FEAT), lambda i: (i, 0)),
            pl.BlockSpec((NC, rows, D_FEAT), lambda i: (0, i, 0)),
        ],
        out_specs=pl.BlockSpec((rows, D_FEAT), lambda i: (i, 0)),
        out_shape=jax.ShapeDtypeStruct((N_NODES, D_FEAT), jnp.float32),
    )(eps, feat, part)


@jax.jit
def kernel(feat, edge_index, eps):
    src3 = edge_index[0].astype(jnp.int32).reshape(N_CHUNKS, 1, CHUNK)
    dst3 = edge_index[1].astype(jnp.int32).reshape(N_CHUNKS, 1, CHUNK)
    part = _sc_partials(feat, src3, dst3)
    return _tc_combine(eps, feat, part)
